# Initial kernel scaffold; baseline (speedup 1.0000x reference)
import functools

import jax
import jax.numpy as jnp
import numpy as np
from jax import lax
from jax.experimental import pallas as pl
from jax.experimental.pallas import tpu as pltpu
from jax.experimental.pallas import tpu_sc as plsc

NUM_LAYER = 5
DIM = 128
N_NODES = 10000
N_EDGES = 320000

NC = 2
NS = 16
NW = NC * NS

CHUNK = 128
E_PAD = 323584
CHUNKS_PER_W = E_PAD // (NW * CHUNK)
NPAD = 10240
ROWS_PER_TILE = NPAD // NS
NB = 5
RB = N_NODES // NB

_CT = np.zeros((16, 16), np.float32)
for _c in range(9):
    _CT[_c, _c // 3] = 1.0
    _CT[_c, 8 + _c % 3] = 1.0


def _zero_tile_chunk(buf, shared, s, ncols):
    z = jnp.zeros((16,), jnp.float32)

    def zrow(i, carry):
        for j in range(ncols // 16):
            buf[i, pl.ds(j * 16, 16)] = z
        return carry

    lax.fori_loop(0, CHUNK, zrow, 0)
    rbase = s * ROWS_PER_TILE
    for k in range(ROWS_PER_TILE // CHUNK):
        pltpu.sync_copy(buf, shared.at[pl.ds(rbase + k * CHUNK, CHUNK)])


def _edge_agg_body(h_hbm, src_hbm, dst_hbm, out_hbm,
                   agg_sh, sidx, didx, rows, sem):
    c = lax.axis_index("c")
    s = lax.axis_index("s")
    wid = c * NS + s

    _zero_tile_chunk(rows, agg_sh, s, DIM)
    plsc.subcore_barrier()

    ebase = wid * (CHUNKS_PER_W * CHUNK)

    def body(i, carry):
        off = ebase + i * CHUNK
        pltpu.sync_copy(src_hbm.at[pl.ds(off, CHUNK)], sidx)
        pltpu.async_copy(h_hbm.at[sidx], rows, sem).wait()
        pltpu.sync_copy(dst_hbm.at[pl.ds(off, CHUNK)], didx)
        pltpu.sync_copy(rows, agg_sh.at[didx], add=True)
        return carry

    lax.fori_loop(0, CHUNKS_PER_W, body, 0)
    plsc.subcore_barrier()

    rbase = s * ROWS_PER_TILE
    obase = c * NPAD + rbase
    for k in range(ROWS_PER_TILE // CHUNK):
        pltpu.sync_copy(agg_sh.at[pl.ds(rbase + k * CHUNK, CHUNK)], rows)
        pltpu.sync_copy(rows, out_hbm.at[pl.ds(obase + k * CHUNK, CHUNK)])


def _counts_body(tbl_hbm, ea0_hbm, ea1_hbm, dst_hbm, out_hbm,
                 cnt_sh, aidx, bidx, cidx, didx, crows, sem):
    c = lax.axis_index("c")
    s = lax.axis_index("s")
    wid = c * NS + s

    _zero_tile_chunk(crows, cnt_sh, s, 16)
    plsc.subcore_barrier()

    ebase = wid * (CHUNKS_PER_W * CHUNK)

    def body(i, carry):
        off = ebase + i * CHUNK
        pltpu.sync_copy(ea0_hbm.at[pl.ds(off, CHUNK)], aidx)
        pltpu.sync_copy(ea1_hbm.at[pl.ds(off, CHUNK)], bidx)
        for j in range(CHUNK // 16):
            av = aidx[pl.ds(j * 16, 16)]
            bv = bidx[pl.ds(j * 16, 16)]
            cidx[pl.ds(j * 16, 16)] = av * 3 + bv
        pltpu.async_copy(tbl_hbm.at[cidx], crows, sem).wait()
        pltpu.sync_copy(dst_hbm.at[pl.ds(off, CHUNK)], didx)
        pltpu.sync_copy(crows, cnt_sh.at[didx], add=True)
        return carry

    lax.fori_loop(0, CHUNKS_PER_W, body, 0)
    plsc.subcore_barrier()

    rbase = s * ROWS_PER_TILE
    obase = c * NPAD + rbase
    for k in range(ROWS_PER_TILE // CHUNK):
        pltpu.sync_copy(cnt_sh.at[pl.ds(rbase + k * CHUNK, CHUNK)], crows)
        pltpu.sync_copy(crows, out_hbm.at[pl.ds(obase + k * CHUNK, CHUNK)])


@functools.lru_cache(maxsize=None)
def _sc_kernels():
    mesh = plsc.VectorSubcoreMesh(
        core_axis_name="c", subcore_axis_name="s",
        num_cores=NC, num_subcores=NS)
    f32 = jnp.float32
    edge_agg = pl.kernel(
        _edge_agg_body,
        out_type=jax.ShapeDtypeStruct((NC * NPAD, DIM), f32),
        mesh=mesh,
        scratch_types=[
            pltpu.VMEM_SHARED((NPAD, DIM), f32),
            pltpu.VMEM((CHUNK,), jnp.int32),
            pltpu.VMEM((CHUNK,), jnp.int32),
            pltpu.VMEM((CHUNK, DIM), f32),
            pltpu.SemaphoreType.DMA,
        ],
    )
    counts = pl.kernel(
        _counts_body,
        out_type=jax.ShapeDtypeStruct((NC * NPAD, 16), f32),
        mesh=mesh,
        scratch_types=[
            pltpu.VMEM_SHARED((NPAD, 16), f32),
            pltpu.VMEM((CHUNK,), jnp.int32),
            pltpu.VMEM((CHUNK,), jnp.int32),
            pltpu.VMEM((CHUNK,), jnp.int32),
            pltpu.VMEM((CHUNK,), jnp.int32),
            pltpu.VMEM((CHUNK, 16), f32),
            pltpu.SemaphoreType.DMA,
        ],
    )
    return edge_agg, counts


def _lin_body(x_ref, w_ref, b_ref, o_ref):
    o_ref[...] = jnp.maximum(x_ref[...] @ w_ref[...] + b_ref[...], 0.0)


def _mlp_body(p_ref, h_ref, cnt_ref, t_ref, eb_ref,
              w1_ref, b1_ref, w2_ref, b2_ref, o_ref, st_ref):
    i = pl.program_id(0)
    cnt = cnt_ref[0] + cnt_ref[1]
    agg = (p_ref[0] + p_ref[1] + h_ref[...]
           + cnt @ t_ref[...] + eb_ref[...])
    hid = jnp.maximum(agg @ w1_ref[...] + b1_ref[...], 0.0)
    out = hid @ w2_ref[...] + b2_ref[...]
    o_ref[...] = out
    blk = jnp.stack([jnp.sum(out, axis=0), jnp.sum(out * out, axis=0)])

    @pl.when(i == 0)
    def _():
        st_ref[...] = blk

    @pl.when(i > 0)
    def _():
        st_ref[...] += blk


def _bn_body(o_ref, st_ref, g_ref, be_ref, h_ref):
    inv_n = 1.0 / N_NODES
    mu = st_ref[0:1, :] * inv_n
    var = st_ref[1:2, :] * inv_n - mu * mu
    h_ref[...] = (g_ref[...] * (o_ref[...] - mu) * lax.rsqrt(var + 1e-5)
                  + be_ref[...])


@functools.lru_cache(maxsize=None)
def _tc_kernels():
    f32 = jnp.float32
    lin = pl.pallas_call(
        _lin_body,
        out_shape=jax.ShapeDtypeStruct((N_NODES, DIM), f32),
    )
    mlp = pl.pallas_call(
        _mlp_body,
        grid=(NB,),
        in_specs=[
            pl.BlockSpec((NC, RB, DIM), lambda i: (0, i, 0)),
            pl.BlockSpec((RB, DIM), lambda i: (i, 0)),
            pl.BlockSpec((NC, RB, 16), lambda i: (0, i, 0)),
            pl.BlockSpec((16, DIM), lambda i: (0, 0)),
            pl.BlockSpec((1, DIM), lambda i: (0, 0)),
            pl.BlockSpec((DIM, 2 * DIM), lambda i: (0, 0)),
            pl.BlockSpec((1, 2 * DIM), lambda i: (0, 0)),
            pl.BlockSpec((2 * DIM, DIM), lambda i: (0, 0)),
            pl.BlockSpec((1, DIM), lambda i: (0, 0)),
        ],
        out_specs=[
            pl.BlockSpec((RB, DIM), lambda i: (i, 0)),
            pl.BlockSpec((2, DIM), lambda i: (0, 0)),
        ],
        out_shape=[
            jax.ShapeDtypeStruct((N_NODES, DIM), f32),
            jax.ShapeDtypeStruct((2, DIM), f32),
        ],
    )
    bn = pl.pallas_call(
        _bn_body,
        grid=(NB,),
        in_specs=[
            pl.BlockSpec((RB, DIM), lambda i: (i, 0)),
            pl.BlockSpec((2, DIM), lambda i: (0, 0)),
            pl.BlockSpec((1, DIM), lambda i: (0, 0)),
            pl.BlockSpec((1, DIM), lambda i: (0, 0)),
        ],
        out_specs=pl.BlockSpec((RB, DIM), lambda i: (i, 0)),
        out_shape=jax.ShapeDtypeStruct((N_NODES, DIM), f32),
    )
    return lin, mlp, bn


def kernel(x, edge_index, edge_attr, lin_W, lin_b, W1, b1, W2, b2,
           emb1, emb2, gamma, beta):
    f32 = jnp.float32
    i32 = jnp.int32
    pad = E_PAD - N_EDGES

    src_p = jnp.concatenate([edge_index[0], jnp.zeros((pad,), i32)])
    dst_p = jnp.concatenate([edge_index[1], jnp.full((pad,), N_NODES, i32)])
    ea0_p = jnp.concatenate([edge_attr[:, 0], jnp.full((pad,), 5, i32)])
    ea1_p = jnp.concatenate([edge_attr[:, 1], jnp.zeros((pad,), i32)])
    tbl = jnp.asarray(_CT)

    edge_agg, counts_k = _sc_kernels()
    lin, mlp, bn = _tc_kernels()

    h = lin(x, lin_W, lin_b.reshape(1, DIM))
    cnt = counts_k(tbl, ea0_p, ea1_p, dst_p).reshape(NC, NPAD, 16)

    for l in range(NUM_LAYER):
        t_l = jnp.concatenate(
            [emb1[l], jnp.zeros((2, DIM), f32), emb2[l],
             jnp.zeros((5, DIM), f32)], axis=0)
        eb_l = (emb1[l, 4] + emb2[l, 0]).reshape(1, DIM)
        p = edge_agg(h, src_p, dst_p).reshape(NC, NPAD, DIM)
        out, st = mlp(p, h, cnt, t_l, eb_l,
                      W1[l], b1[l].reshape(1, 2 * DIM),
                      W2[l], b2[l].reshape(1, DIM))
        h = bn(out, st, gamma[l].reshape(1, DIM), beta[l].reshape(1, DIM))
    return h


# speedup vs baseline: 4.0976x; 4.0976x over previous
import functools

import jax
import jax.numpy as jnp
import numpy as np
from jax import lax
from jax.experimental import pallas as pl
from jax.experimental.pallas import tpu as pltpu
from jax.experimental.pallas import tpu_sc as plsc

NUM_LAYER = 5
DIM = 128
N_NODES = 10000
N_EDGES = 320000

NC = 2
NS = 16
NW = NC * NS

CHUNK = 128
E_PAD = 323584
CHUNKS_PER_W = E_PAD // (NW * CHUNK)
NPAD = 10240
ROWS_PER_TILE = NPAD // NS
NB = 5
RB = N_NODES // NB

_CT = np.zeros((16, DIM), np.float32)
for _c in range(9):
    _CT[_c, _c // 3] = 1.0
    _CT[_c, 8 + _c % 3] = 1.0


def _zero_tile_chunk(buf, shared, s, ncols):
    z = jnp.zeros((16,), jnp.float32)

    def zrow(i, carry):
        for j in range(ncols // 16):
            buf[i, pl.ds(j * 16, 16)] = z
        return carry

    lax.fori_loop(0, CHUNK, zrow, 0)
    rbase = s * ROWS_PER_TILE
    for k in range(ROWS_PER_TILE // CHUNK):
        pltpu.sync_copy(buf, shared.at[pl.ds(rbase + k * CHUNK, CHUNK)])


def _edge_agg_body(h_hbm, src_hbm, dst_hbm, out_hbm,
                   agg_sh, sidx, didx, rows, sem):
    c = lax.axis_index("c")
    s = lax.axis_index("s")
    wid = c * NS + s

    _zero_tile_chunk(rows, agg_sh, s, DIM)
    plsc.subcore_barrier()

    ebase = wid * (CHUNKS_PER_W * CHUNK)

    def body(i, carry):
        off = ebase + i * CHUNK
        pltpu.sync_copy(src_hbm.at[pl.ds(off, CHUNK)], sidx)
        pltpu.async_copy(h_hbm.at[sidx], rows, sem).wait()
        pltpu.sync_copy(dst_hbm.at[pl.ds(off, CHUNK)], didx)
        pltpu.sync_copy(rows, agg_sh.at[didx], add=True)
        return carry

    lax.fori_loop(0, CHUNKS_PER_W, body, 0)
    plsc.subcore_barrier()

    rbase = s * ROWS_PER_TILE
    obase = c * NPAD + rbase
    for k in range(ROWS_PER_TILE // CHUNK):
        pltpu.sync_copy(agg_sh.at[pl.ds(rbase + k * CHUNK, CHUNK)], rows)
        pltpu.sync_copy(rows, out_hbm.at[pl.ds(obase + k * CHUNK, CHUNK)])


def _counts_body(tbl_hbm, ea0_hbm, ea1_hbm, dst_hbm, out_hbm,
                 cnt_sh, aidx, bidx, cidx, didx, crows, sem):
    c = lax.axis_index("c")
    s = lax.axis_index("s")
    wid = c * NS + s

    _zero_tile_chunk(crows, cnt_sh, s, DIM)
    plsc.subcore_barrier()

    ebase = wid * (CHUNKS_PER_W * CHUNK)

    def body(i, carry):
        off = ebase + i * CHUNK
        pltpu.sync_copy(ea0_hbm.at[pl.ds(off, CHUNK)], aidx)
        pltpu.sync_copy(ea1_hbm.at[pl.ds(off, CHUNK)], bidx)
        for j in range(CHUNK // 16):
            av = aidx[pl.ds(j * 16, 16)]
            bv = bidx[pl.ds(j * 16, 16)]
            cidx[pl.ds(j * 16, 16)] = av * 3 + bv
        pltpu.async_copy(tbl_hbm.at[cidx], crows, sem).wait()
        pltpu.sync_copy(dst_hbm.at[pl.ds(off, CHUNK)], didx)
        pltpu.sync_copy(crows, cnt_sh.at[didx], add=True)
        return carry

    lax.fori_loop(0, CHUNKS_PER_W, body, 0)
    plsc.subcore_barrier()

    rbase = s * ROWS_PER_TILE
    obase = c * NPAD + rbase
    for k in range(ROWS_PER_TILE // CHUNK):
        pltpu.sync_copy(cnt_sh.at[pl.ds(rbase + k * CHUNK, CHUNK)], crows)
        pltpu.sync_copy(crows, out_hbm.at[pl.ds(obase + k * CHUNK, CHUNK)])


@functools.lru_cache(maxsize=None)
def _sc_kernels():
    mesh = plsc.VectorSubcoreMesh(
        core_axis_name="c", subcore_axis_name="s",
        num_cores=NC, num_subcores=NS)
    f32 = jnp.float32
    edge_agg = pl.kernel(
        _edge_agg_body,
        out_type=jax.ShapeDtypeStruct((NC * NPAD, DIM), f32),
        mesh=mesh,
        scratch_types=[
            pltpu.VMEM_SHARED((NPAD, DIM), f32),
            pltpu.VMEM((CHUNK,), jnp.int32),
            pltpu.VMEM((CHUNK,), jnp.int32),
            pltpu.VMEM((CHUNK, DIM), f32),
            pltpu.SemaphoreType.DMA,
        ],
    )
    counts = pl.kernel(
        _counts_body,
        out_type=jax.ShapeDtypeStruct((NC * NPAD, DIM), f32),
        mesh=mesh,
        scratch_types=[
            pltpu.VMEM_SHARED((NPAD, DIM), f32),
            pltpu.VMEM((CHUNK,), jnp.int32),
            pltpu.VMEM((CHUNK,), jnp.int32),
            pltpu.VMEM((CHUNK,), jnp.int32),
            pltpu.VMEM((CHUNK,), jnp.int32),
            pltpu.VMEM((CHUNK, DIM), f32),
            pltpu.SemaphoreType.DMA,
        ],
    )
    return edge_agg, counts


def _lin_body(x_ref, w_ref, b_ref, o_ref):
    o_ref[...] = jnp.maximum(x_ref[...] @ w_ref[...] + b_ref[...], 0.0)


def _mlp_body(p_ref, h_ref, cnt_ref, t_ref, eb_ref,
              w1_ref, b1_ref, w2_ref, b2_ref, o_ref, st_ref):
    i = pl.program_id(0)
    cnt = cnt_ref[0][:, :16] + cnt_ref[1][:, :16]
    emb_agg = jnp.dot(cnt, t_ref[...], precision=lax.Precision.HIGHEST)
    agg = p_ref[0] + p_ref[1] + h_ref[...] + emb_agg + eb_ref[...]
    hid = jnp.maximum(agg @ w1_ref[...] + b1_ref[...], 0.0)
    out = hid @ w2_ref[...] + b2_ref[...]
    o_ref[...] = out
    blk = jnp.stack([jnp.sum(out, axis=0), jnp.sum(out * out, axis=0)])

    @pl.when(i == 0)
    def _():
        st_ref[...] = blk

    @pl.when(i > 0)
    def _():
        st_ref[...] += blk


def _bn_body(o_ref, st_ref, g_ref, be_ref, h_ref):
    inv_n = 1.0 / N_NODES
    mu = st_ref[0:1, :] * inv_n
    var = st_ref[1:2, :] * inv_n - mu * mu
    h_ref[...] = (g_ref[...] * (o_ref[...] - mu) * lax.rsqrt(var + 1e-5)
                  + be_ref[...])


@functools.lru_cache(maxsize=None)
def _tc_kernels():
    f32 = jnp.float32
    lin = pl.pallas_call(
        _lin_body,
        out_shape=jax.ShapeDtypeStruct((N_NODES, DIM), f32),
    )
    mlp = pl.pallas_call(
        _mlp_body,
        grid=(NB,),
        in_specs=[
            pl.BlockSpec((NC, RB, DIM), lambda i: (0, i, 0)),
            pl.BlockSpec((RB, DIM), lambda i: (i, 0)),
            pl.BlockSpec((NC, RB, DIM), lambda i: (0, i, 0)),
            pl.BlockSpec((16, DIM), lambda i: (0, 0)),
            pl.BlockSpec((1, DIM), lambda i: (0, 0)),
            pl.BlockSpec((DIM, 2 * DIM), lambda i: (0, 0)),
            pl.BlockSpec((1, 2 * DIM), lambda i: (0, 0)),
            pl.BlockSpec((2 * DIM, DIM), lambda i: (0, 0)),
            pl.BlockSpec((1, DIM), lambda i: (0, 0)),
        ],
        out_specs=[
            pl.BlockSpec((RB, DIM), lambda i: (i, 0)),
            pl.BlockSpec((2, DIM), lambda i: (0, 0)),
        ],
        out_shape=[
            jax.ShapeDtypeStruct((N_NODES, DIM), f32),
            jax.ShapeDtypeStruct((2, DIM), f32),
        ],
    )
    bn = pl.pallas_call(
        _bn_body,
        grid=(NB,),
        in_specs=[
            pl.BlockSpec((RB, DIM), lambda i: (i, 0)),
            pl.BlockSpec((2, DIM), lambda i: (0, 0)),
            pl.BlockSpec((1, DIM), lambda i: (0, 0)),
            pl.BlockSpec((1, DIM), lambda i: (0, 0)),
        ],
        out_specs=pl.BlockSpec((RB, DIM), lambda i: (i, 0)),
        out_shape=jax.ShapeDtypeStruct((N_NODES, DIM), f32),
    )
    return lin, mlp, bn


def kernel(x, edge_index, edge_attr, lin_W, lin_b, W1, b1, W2, b2,
           emb1, emb2, gamma, beta):
    f32 = jnp.float32
    i32 = jnp.int32
    pad = E_PAD - N_EDGES

    src_p = jnp.concatenate([edge_index[0], jnp.zeros((pad,), i32)])
    dst_p = jnp.concatenate([edge_index[1], jnp.full((pad,), N_NODES, i32)])
    ea0_p = jnp.concatenate([edge_attr[:, 0], jnp.full((pad,), 5, i32)])
    ea1_p = jnp.concatenate([edge_attr[:, 1], jnp.zeros((pad,), i32)])
    tbl = jnp.asarray(_CT)

    edge_agg, counts_k = _sc_kernels()
    lin, mlp, bn = _tc_kernels()

    h = lin(x, lin_W, lin_b.reshape(1, DIM))
    cnt = counts_k(tbl, ea0_p, ea1_p, dst_p).reshape(NC, NPAD, DIM)

    for l in range(NUM_LAYER):
        t_l = jnp.concatenate(
            [emb1[l], jnp.zeros((2, DIM), f32), emb2[l],
             jnp.zeros((5, DIM), f32)], axis=0)
        eb_l = (emb1[l, 4] + emb2[l, 0]).reshape(1, DIM)
        p = edge_agg(h, src_p, dst_p).reshape(NC, NPAD, DIM)
        out, st = mlp(p, h, cnt, t_l, eb_l,
                      W1[l], b1[l].reshape(1, 2 * DIM),
                      W2[l], b2[l].reshape(1, DIM))
        h = bn(out, st, gamma[l].reshape(1, DIM), beta[l].reshape(1, DIM))
    return h
